# Initial kernel scaffold; baseline (speedup 1.0000x reference)
import jax
import jax.numpy as jnp
from jax.experimental import pallas as pl


def kernel(x1, edge_index1, batch1, x2, edge_index2, batch2, W1, b1, W2, b2, Wf1, bf1, Wf2, bf2, Wfc1, bfc1):
    raise NotImplementedError("write your pallas kernel here")


# speedup vs baseline: 5.3327x; 5.3327x over previous
import functools
from functools import partial

import jax
import jax.numpy as jnp
from jax import lax
from jax.experimental import pallas as pl
from jax.experimental.pallas import tpu as pltpu, tpu_sc as plsc

NUM_NET = 5
G = 64
OUT1 = 64
OUT2 = 16
NSC = 2
NT = 16
LK = 128

_mesh = lambda: plsc.VectorSubcoreMesh(
    core_axis_name="c", subcore_axis_name="s", num_cores=NSC, num_subcores=NT)


def _ceil_to(x, m):
  return (x + m - 1) // m * m


def _make_prop_kernel(nnets, n, dh, e_pad, acc_rows):
  zrows = acc_rows // NT
  wrows = n // NT
  chunks = e_pad // (NT * LK)

  @functools.partial(
      pl.kernel,
      out_type=jax.ShapeDtypeStruct((nnets, n, NSC, dh), jnp.float32),
      mesh=_mesh(),
      scratch_types=dict(
          srcv=pltpu.VMEM((chunks, LK), jnp.int32),
          dba=pltpu.VMEM((1, LK), jnp.int32),
          dbb=pltpu.VMEM((1, LK), jnp.int32),
          bufa=pltpu.VMEM((LK, dh), jnp.float32),
          bufb=pltpu.VMEM((LK, dh), jnp.float32),
          sema=pltpu.SemaphoreType.DMA,
          semb=pltpu.SemaphoreType.DMA,
          semia=pltpu.SemaphoreType.DMA,
          semib=pltpu.SemaphoreType.DMA,
          acc=pltpu.VMEM_SHARED((acc_rows, dh), jnp.float32),
      ),
      compiler_params=pltpu.CompilerParams(use_tc_tiling_on_sc=False),
  )
  def prop_kernel(srcl_hbm, srch_hbm, dst_hbm, v_hbm, zeros_hbm, out_hbm,
                  srcv, dba, dbb, bufa, bufb, sema, semb, semia, semib, acc):
    c = lax.axis_index("c")
    s = lax.axis_index("s")
    base = s * chunks

    @pl.when(c == 0)
    def _():
      pltpu.sync_copy(srcl_hbm.at[pl.ds(base, chunks)], srcv)

    @pl.when(c == 1)
    def _():
      pltpu.sync_copy(srch_hbm.at[pl.ds(base, chunks)], srcv)

    for i in range(nnets):
      table = v_hbm.at[i]
      pltpu.sync_copy(zeros_hbm, acc.at[pl.ds(s * zrows, zrows)])
      plsc.subcore_barrier()
      pltpu.async_copy(dst_hbm.at[pl.ds(base, 1)], dba, semia)
      pltpu.async_copy(table.at[srcv.at[0]], bufa, sema)
      pltpu.async_copy(dst_hbm.at[pl.ds(base + 1, 1)], dbb, semib)
      pltpu.async_copy(table.at[srcv.at[1]], bufb, semb)

      def body(g, carry):
        ja = 2 * g
        pltpu.make_async_copy(dst_hbm.at[pl.ds(base, 1)], dba, semia).wait()
        pltpu.make_async_copy(table.at[srcv.at[ja]], bufa, sema).wait()
        pltpu.sync_copy(bufa, acc.at[dba.at[0]], add=True)

        @pl.when(ja + 2 < chunks)
        def _():
          pltpu.async_copy(dst_hbm.at[pl.ds(base + ja + 2, 1)], dba, semia)
          pltpu.async_copy(table.at[srcv.at[ja + 2]], bufa, sema)

        pltpu.make_async_copy(dst_hbm.at[pl.ds(base, 1)], dbb, semib).wait()
        pltpu.make_async_copy(table.at[srcv.at[ja + 1]], bufb, semb).wait()
        pltpu.sync_copy(bufb, acc.at[dbb.at[0]], add=True)

        @pl.when(ja + 3 < chunks)
        def _():
          pltpu.async_copy(dst_hbm.at[pl.ds(base + ja + 3, 1)], dbb, semib)
          pltpu.async_copy(table.at[srcv.at[ja + 3]], bufb, semb)
        return carry

      lax.fori_loop(0, chunks // 2, body, 0)
      plsc.subcore_barrier()
      pltpu.sync_copy(acc.at[pl.ds(s * wrows, wrows)],
                      out_hbm.at[i, pl.ds(s * wrows, wrows), c])

  return prop_kernel


def _prep_block(degp, x, dinv_o, v0_o):
  deg = degp[0, :, 0:1] + degp[1, :, 0:1] + 1.0
  dinv = lax.rsqrt(deg)
  dinv_o[...] = dinv
  v0_o[...] = x[...] * dinv


def _run_prep(degp, x, bn):
  n, d = x.shape
  nb = n // bn
  return pl.pallas_call(
      _prep_block,
      grid=(nb,),
      in_specs=[
          pl.BlockSpec((NSC, bn, 16), lambda r: (0, r, 0)),
          pl.BlockSpec((bn, d), lambda r: (r, 0)),
      ],
      out_specs=[
          pl.BlockSpec((bn, 1), lambda r: (r, 0)),
          pl.BlockSpec((bn, d), lambda r: (r, 0)),
      ],
      out_shape=[
          jax.ShapeDtypeStruct((n, 1), jnp.float32),
          jax.ShapeDtypeStruct((n, d), jnp.float32),
      ],
  )(degp, x)


def _gcn_mm_block(s, v, dinv, w, b, o, *, last):
  di = dinv[...]
  a = (s[0] + v[0]) * di
  y = jnp.dot(a, w[0], preferred_element_type=jnp.float32) + b[0]
  if last:
    o[0] = y
  else:
    o[0] = jnp.maximum(y, 0.0) * di


def _run_gcn_mm(s, v, dinv, w, b, bn, last):
  b = b.reshape(NUM_NET, 1, -1)
  sn = s.shape[0]
  vn = v.shape[0]
  n, d = s.shape[1], s.shape[2]
  nb = n // bn
  return pl.pallas_call(
      functools.partial(_gcn_mm_block, last=last),
      grid=(NUM_NET, nb),
      in_specs=[
          pl.BlockSpec((1, bn, d), (lambda i, r: (i, r, 0)) if sn > 1 else (lambda i, r: (0, r, 0))),
          pl.BlockSpec((1, bn, d), (lambda i, r: (i, r, 0)) if vn > 1 else (lambda i, r: (0, r, 0))),
          pl.BlockSpec((bn, 1), lambda i, r: (r, 0)),
          pl.BlockSpec((1, d, d), lambda i, r: (i, 0, 0)),
          pl.BlockSpec((1, 1, d), lambda i, r: (i, 0, 0)),
      ],
      out_specs=pl.BlockSpec((1, bn, d), lambda i, r: (i, r, 0)),
      out_shape=jax.ShapeDtypeStruct((NUM_NET, n, d), jnp.float32),
  )(s, v, dinv, w, b)


def _pool_block(h, bseg, o, scratch):
  r = pl.program_id(1)

  @pl.when(r == 0)
  def _():
    o[...] = jnp.full(o.shape, -jnp.inf, o.dtype)

  rows = h[0]
  b = bseg[...]
  bn = rows.shape[0]
  k = 1
  while k < bn:
    rs = jnp.concatenate([jnp.full((k,) + rows.shape[1:], -jnp.inf, rows.dtype),
                          rows[:-k]], axis=0)
    bs = jnp.concatenate([jnp.full((k, 1), -1, b.dtype), b[:-k]], axis=0)
    rows = jnp.where(bs == b, jnp.maximum(rows, rs), rows)
    k *= 2
  scratch[...] = rows
  for g in range(G):
    le = jnp.sum((b <= g).astype(jnp.int32))
    lt = jnp.sum((b < g).astype(jnp.int32))
    idx = jnp.maximum(le - 1, 0)
    rowg = scratch[pl.ds(idx, 1), :]
    val = jnp.where(le > lt, rowg, -jnp.inf)
    o[0, pl.ds(g, 1), :] = jnp.maximum(o[0, pl.ds(g, 1), :], val)


def _run_pool(h, bseg, bn):
  nn, n, d = h.shape
  nb = n // bn
  return pl.pallas_call(
      _pool_block,
      grid=(nn, nb),
      in_specs=[
          pl.BlockSpec((1, bn, d), lambda i, r: (i, r, 0)),
          pl.BlockSpec((bn, 1), lambda i, r: (r, 0)),
      ],
      out_specs=pl.BlockSpec((1, G, d), lambda i, r: (i, 0, 0)),
      out_shape=jax.ShapeDtypeStruct((nn, G, d), jnp.float32),
      scratch_shapes=[pltpu.VMEM((bn, d), jnp.float32)],
  )(h, bseg)


def _heads_block(px1, ph1, px2, ph2, wf1, bf1, wf2, bf2, wfc, bfc, o):
  d1 = px1.shape[2]
  d2 = px2.shape[2]
  acc = jnp.zeros((G, 64), jnp.float32)
  for i in range(NUM_NET):
    z = jnp.dot(px1[0], wf1[i, :d1], preferred_element_type=jnp.float32)
    z = z + jnp.dot(ph1[i], wf1[i, d1:], preferred_element_type=jnp.float32)
    z = jnp.maximum(z + bf1[i], 0.0)
    acc = acc + jnp.dot(z, wfc[pl.ds(i * OUT1, OUT1), :],
                        preferred_element_type=jnp.float32)
  off = NUM_NET * OUT1
  for i in range(NUM_NET):
    z = jnp.dot(px2[0], wf2[i, :d2], preferred_element_type=jnp.float32)
    z = z + jnp.dot(ph2[i], wf2[i, d2:], preferred_element_type=jnp.float32)
    z = jnp.maximum(z + bf2[i], 0.0)
    acc = acc + jnp.dot(z, wfc[pl.ds(off + i * OUT2, OUT2), :],
                        preferred_element_type=jnp.float32)
  o[...] = acc + bfc[...]


def _run_heads(px1, ph1, px2, ph2, wf1, bf1, wf2, bf2, wfc, bfc):
  spec = lambda a: pl.BlockSpec(a.shape, lambda: (0,) * a.ndim)
  args = (px1, ph1, px2, ph2, wf1, bf1, wf2, bf2, wfc, bfc)
  return pl.pallas_call(
      _heads_block,
      in_specs=[spec(a) for a in args],
      out_specs=pl.BlockSpec((G, 64), lambda: (0, 0)),
      out_shape=jax.ShapeDtypeStruct((G, 64), jnp.float32),
  )(*args)


def _branch_pooled(x, edge_index, batch, W, b, acc_rows, bn):
  n, d = x.shape
  dh = d // 2
  e = edge_index.shape[1]
  e_pad = _ceil_to(e, NSC * NT * LK)
  src = edge_index[0].astype(jnp.int32)
  dst = edge_index[1].astype(jnp.int32)
  pad = e_pad - e
  dummy = jnp.int32(n)
  srcp = jnp.concatenate([src, jnp.zeros((pad,), jnp.int32)])
  dstp = jnp.concatenate([dst, jnp.full((pad,), dummy, jnp.int32)])
  srcl = (2 * srcp).reshape(-1, LK)
  srch = (2 * srcp + 1).reshape(-1, LK)
  dst2 = dstp.reshape(-1, LK)

  ones16 = jnp.ones((LK, 16), jnp.float32)
  zeros16 = jnp.zeros((acc_rows // NT, 16), jnp.float32)
  zerosdh = jnp.zeros((acc_rows // NT, dh), jnp.float32)

  deg_e_pad = _ceil_to(e, NSC * NT * LK)
  degp = _deg(dst2, ones16, zeros16, e_pad=deg_e_pad, acc_rows=acc_rows)
  dinv, v0 = _run_prep(degp, x, bn)

  prop = _make_prop_kernel(1, n, dh, e_pad, acc_rows)
  prop5 = _make_prop_kernel(NUM_NET, n, dh, e_pad, acc_rows)

  s0 = prop(srcl, srch, dst2, v0.reshape(1, 2 * n, dh), zerosdh)
  s0 = s0.reshape(1, n, d)
  v1 = _run_gcn_mm(s0, v0.reshape(1, n, d), dinv, W[:, 0], b[:, 0], bn, last=False)
  s1 = prop5(srcl, srch, dst2, v1.reshape(NUM_NET, 2 * n, dh), zerosdh)
  s1 = s1.reshape(NUM_NET, n, d)
  v2 = _run_gcn_mm(s1, v1, dinv, W[:, 1], b[:, 1], bn, last=False)
  s2 = prop5(srcl, srch, dst2, v2.reshape(NUM_NET, 2 * n, dh), zerosdh)
  s2 = s2.reshape(NUM_NET, n, d)
  h3 = _run_gcn_mm(s2, v2, dinv, W[:, 2], b[:, 2], bn, last=True)

  bseg = batch.astype(jnp.int32).reshape(n, 1)
  px = _run_pool(x.reshape(1, n, d), bseg, bn)
  ph = _run_pool(h3, bseg, bn)
  return px, ph


def _deg(dst2, ones16, zeros16, *, e_pad, acc_rows):
  rows_per_tile = acc_rows // NT
  chunks_per_tile = e_pad // (NSC * NT * LK)

  @functools.partial(
      pl.kernel,
      out_type=jax.ShapeDtypeStruct((NSC, acc_rows, 16), jnp.float32),
      mesh=_mesh(),
      scratch_types=dict(
          dstv=pltpu.VMEM((chunks_per_tile, LK), jnp.int32),
          onesv=pltpu.VMEM((LK, 16), jnp.float32),
          zv=pltpu.VMEM((rows_per_tile, 16), jnp.float32),
          acc=pltpu.VMEM_SHARED((acc_rows, 16), jnp.float32),
      ),
      compiler_params=pltpu.CompilerParams(use_tc_tiling_on_sc=False),
  )
  def deg_kernel(dst_hbm, ones_hbm, zeros_hbm, out_hbm, dstv, onesv, zv, acc):
    c = lax.axis_index("c")
    s = lax.axis_index("s")
    w = c * NT + s
    pltpu.sync_copy(dst_hbm.at[pl.ds(w * chunks_per_tile, chunks_per_tile)], dstv)
    pltpu.sync_copy(ones_hbm, onesv)
    pltpu.sync_copy(zeros_hbm, zv)
    pltpu.sync_copy(zv, acc.at[pl.ds(s * rows_per_tile, rows_per_tile)])
    plsc.subcore_barrier()
    for j in range(chunks_per_tile):
      pltpu.sync_copy(onesv, acc.at[dstv.at[j]], add=True)
    plsc.subcore_barrier()
    pltpu.sync_copy(acc.at[pl.ds(s * rows_per_tile, rows_per_tile)],
                    out_hbm.at[c, pl.ds(s * rows_per_tile, rows_per_tile)])

  return deg_kernel(dst2, ones16, zeros16)


def kernel(x1, edge_index1, batch1, x2, edge_index2, batch2,
           W1, b1, W2, b2, Wf1, bf1, Wf2, bf2, Wfc1, bfc1):
  n1 = x1.shape[0]
  n2 = x2.shape[0]
  acc1 = _ceil_to(n1 + 16, NT * 16)
  acc2 = _ceil_to(n2 + 16, NT * 16)
  px1, ph1 = _branch_pooled(x1, edge_index1, batch1, W1, b1, acc1, 2000)
  px2, ph2 = _branch_pooled(x2, edge_index2, batch2, W2, b2, acc2, 2000)
  return _run_heads(px1[0:1].reshape(1, G, x1.shape[1]), ph1,
                    px2[0:1].reshape(1, G, x2.shape[1]), ph2,
                    Wf1, bf1, Wf2, bf2, Wfc1, bfc1)
